# baseline (speedup 1.0000x reference)
import functools
import jax
import jax.numpy as jnp
from jax import lax
from jax.experimental import pallas as pl
from jax.experimental import pallas as _pl_unused
from jax.experimental.pallas import tpu as pltpu
from jax.experimental.pallas import tpu_sc as plsc

N = 10000
E = 320000
D = 128
DP = D // 2
L = 16
NC = 2
NS = 16
NW = NC * NS
NPW = 320
NPAD = NW * NPW
C = 8000
NCHUNK = E // C
G = 64
NSLOT = 4
MBUF = C + 192
DUMP = NPW
NEGP = -8421505

_mesh = plsc.VectorSubcoreMesh(
    core_axis_name="c", subcore_axis_name="s", num_cores=NC, num_subcores=NS
)


@functools.partial(
    pl.kernel,
    out_type=jax.ShapeDtypeStruct((NPAD * D,), jnp.float32),
    mesh=_mesh,
    compiler_params=pltpu.CompilerParams(needs_layout_passes=False),
    scratch_types=[
        pltpu.VMEM(((NPW + 1) * D,), jnp.float32),
        pltpu.VMEM(((NPW + 1) * DP,), jnp.int32),
        pltpu.VMEM((NPW + 2 * L,), jnp.float32),
        pltpu.VMEM((C,), jnp.int32),
        pltpu.VMEM((C,), jnp.int32),
        pltpu.VMEM((MBUF,), jnp.int32),
        pltpu.VMEM((NSLOT * G,), jnp.int32),
        pltpu.VMEM((NSLOT * G, D), jnp.float32),
        pltpu.VMEM((L,), jnp.float32),
        pltpu.SemaphoreType.DMA,
        pltpu.SemaphoreType.DMA,
        pltpu.SemaphoreType.DMA,
        pltpu.SemaphoreType.DMA,
        pltpu.SemaphoreType.DMA,
        pltpu.SemaphoreType.DMA,
    ],
)
def _mp(z_hbm, ep_hbm, w_hbm, out_hbm,
        s_acc, m_acc, cnt_acc, chA, chB, mpack, gidx, rows, zv,
        semA, semB, sem0, sem1, sem2, sem3):
    sems = [sem0, sem1, sem2, sem3]
    cid = lax.axis_index("c")
    sid = lax.axis_index("s")
    wid = cid * NS + sid
    lo = wid * NPW

    fzeros = jnp.zeros((L,), jnp.float32)
    fones = jnp.ones((L,), jnp.float32)
    izeros = jnp.zeros((L,), jnp.int32)
    iones = jnp.ones((L,), jnp.int32)
    negp = jnp.full((L,), NEGP, jnp.int32)
    dumpv = jnp.full((L,), DUMP << 14, jnp.int32)
    lowv = jnp.full((L,), 16383, jnp.int32)
    highm = jnp.full((L,), -65536, jnp.int32)
    sh16 = jnp.full((L,), 16, jnp.int32)
    sh14 = jnp.full((L,), 14, jnp.int32)
    lanes = lax.iota(jnp.int32, L)
    e1vec = jnp.where(lanes == izeros, fones, fzeros)
    lov14 = jnp.full((L,), lo * 16384, jnp.int32)
    hiv14 = jnp.full((L,), (lo + NPW) * 16384, jnp.int32)

    def init_acc(i, carry):
        s_acc[pl.ds(i * 2 * L, L)] = fzeros
        s_acc[pl.ds(i * 2 * L + L, L)] = fzeros
        m_acc[pl.ds(i * L, L)] = negp
        return carry
    lax.fori_loop(0, (NPW + 1) * DP // L, init_acc, 0)

    def init_cnt(i, carry):
        cnt_acc[pl.ds(i * L, L)] = fzeros
        return carry
    lax.fori_loop(0, (NPW + 2 * L) // L, init_cnt, 0)

    def init_mp(i, carry):
        mpack[pl.ds(i * L, L)] = dumpv
        return carry
    lax.fori_loop(0, MBUF // L, init_mp, 0)

    pltpu.sync_copy(z_hbm, zv)

    def issue_chunk(k, cb, sem):
        pltpu.async_copy(ep_hbm.at[pl.ds(k * C, C)], cb, sem)

    def wait_chunk(cb, sem):
        pltpu.make_async_copy(ep_hbm.at[pl.ds(0, C)], cb, sem).wait()

    def fire(b, s):
        g0 = s * G
        for kk in range(G // L):
            pv = mpack[pl.ds(b * G + kk * L, L)]
            gidx[pl.ds(g0 + kk * L, L)] = pv & lowv
        pltpu.async_copy(
            w_hbm.at[gidx.at[pl.ds(g0, G)]],
            rows.at[pl.ds(g0, G), :], sems[s])

    def drain(s):
        g0 = s * G
        pltpu.make_async_copy(
            w_hbm.at[gidx.at[pl.ds(g0, G)]],
            rows.at[pl.ds(g0, G), :], sems[s]).wait()

    def process(b, s):
        e0 = b * G
        r0 = s * G

        def edge_body(t, carry):
            lsp = mpack[pl.ds(e0 + t, L)][0]
            ls = lax.shift_right_logical(lsp, 14)
            baseS = ls * D
            baseM = ls * DP
            plsc.addupdate(cnt_acc.at[pl.ds(ls, L)], e1vec)
            for h in range(DP // L):
                lof = rows[r0 + t, pl.ds(h * 2 * L, L)]
                hif = rows[r0 + t, pl.ds(h * 2 * L + L, L)]
                plsc.addupdate(s_acc.at[pl.ds(baseS + h * 2 * L, L)], lof)
                plsc.addupdate(s_acc.at[pl.ds(baseS + h * 2 * L + L, L)], hif)
                rb = plsc.pack(lof, hif, format=plsc.PackFormat.INTERLEAVED)
                mw = m_acc[pl.ds(baseM + h * L, L)]
                mx = jnp.maximum(plsc.bitcast(mw, jnp.bfloat16), rb)
                m_acc[pl.ds(baseM + h * L, L)] = plsc.bitcast(mx, jnp.int32)
            return carry
        lax.fori_loop(0, G, edge_body, 0)

    def process_chunk(cb):
        def scan_body(i, off):
            ev = cb[pl.ds(i * L, L)]
            msk = (ev >= lov14) & (ev < hiv14)
            inc = jnp.where(msk, iones, izeros)
            pos = plsc.cumsum(inc)
            idx = jnp.full((L,), off - 1, jnp.int32) + pos
            plsc.store_scatter(mpack, [idx], ev - lov14, mask=msk)
            pc = plsc.all_reduce_population_count(msk)
            return off + pc[0]
        m = lax.fori_loop(0, C // L, scan_body, 0)

        nb = (m + (G - 1)) >> 6
        m64 = nb << 6
        mal = (m >> 4) << 4
        mvec = jnp.full((L,), m, jnp.int32)
        for k in range(G // L + 1):
            ab = mal + k * L

            @pl.when(ab < m64)
            def _():
                v = mpack[pl.ds(ab, L)]
                posv = jnp.full((L,), ab, jnp.int32) + lanes
                mpack[pl.ds(ab, L)] = jnp.where(posv >= mvec, dumpv, v)

        for s in range(NSLOT):
            @pl.when(s < nb)
            def _(s=s):
                fire(s, s)

        nq = (nb + (NSLOT - 1)) >> 2

        def quad_body(q, carry):
            for s in range(NSLOT):
                b = q * NSLOT + s

                @pl.when(b < nb)
                def _(b=b, s=s):
                    drain(s)
                    process(b, s)

                    @pl.when(b + NSLOT < nb)
                    def _():
                        fire(b + NSLOT, s)
            return carry
        lax.fori_loop(0, nq, quad_body, 0)

    issue_chunk(0, chA, semA)

    def chunk_pair(p, carry):
        k0 = 2 * p
        wait_chunk(chA, semA)
        issue_chunk(k0 + 1, chB, semB)
        process_chunk(chA)
        wait_chunk(chB, semB)

        @pl.when(k0 + 2 < NCHUNK)
        def _():
            issue_chunk(k0 + 2, chA, semA)
        process_chunk(chB)
        return carry
    lax.fori_loop(0, NCHUNK // 2, chunk_pair, 0)

    zvec = zv[pl.ds(0, L)]
    z0v = jnp.full((L,), zvec[0])
    z1v = jnp.full((L,), zvec[1])
    z2v = jnp.full((L,), zvec[2])

    def comb_group(ng, carry):
        n0 = ng * L
        cv = cnt_acc[pl.ds(n0, L)]
        scalev = z0v + z1v / jnp.maximum(cv, fones)
        zmxv = jnp.where(cv > fzeros, z2v, fzeros)

        for t in range(L):
            sc = jnp.full((L,), scalev[t])
            zm = jnp.full((L,), zmxv[t])
            baseS = (n0 + t) * D
            baseM = (n0 + t) * DP
            for h in range(DP // L):
                mw = m_acc[pl.ds(baseM + h * L, L)]
                lom = plsc.bitcast(lax.shift_left(mw, sh16), jnp.float32)
                him = plsc.bitcast(mw & highm, jnp.float32)
                slo = s_acc[pl.ds(baseS + h * 2 * L, L)]
                shi = s_acc[pl.ds(baseS + h * 2 * L + L, L)]
                s_acc[pl.ds(baseS + h * 2 * L, L)] = slo * sc + zm * lom
                s_acc[pl.ds(baseS + h * 2 * L + L, L)] = shi * sc + zm * him
        return carry
    lax.fori_loop(0, NPW // L, comb_group, 0)

    pltpu.sync_copy(s_acc.at[pl.ds(0, NPW * D)],
                    out_hbm.at[pl.ds(wid * (NPW * D), NPW * D)])


def kernel(z_agg_hard, edge_index, x):
    z = jnp.pad(z_agg_hard.reshape(3).astype(jnp.float32), (0, L - 3))
    src = edge_index[0].astype(jnp.int32)
    dst = edge_index[1].astype(jnp.int32)
    epack = lax.shift_left(src, 14) | dst
    out = _mp(z, epack, x)
    return out.reshape(NPAD, D)[:N]


# speedup vs baseline: 2.4189x; 2.4189x over previous
import functools
import jax
import jax.numpy as jnp
from jax import lax
from jax.experimental import pallas as pl
from jax.experimental import pallas as _pl_unused
from jax.experimental.pallas import tpu as pltpu
from jax.experimental.pallas import tpu_sc as plsc

N = 10000
E = 320000
D = 128
DP = D // 2
L = 16
NC = 2
NS = 16
NW = NC * NS
NPW = 320
NPAD = NW * NPW
C = 8000
NCHUNK = E // C
G = 64
NSLOT = 2
MBUF = C + 192
DUMP = NPW
NEG = -3.0e38

_mesh = plsc.VectorSubcoreMesh(
    core_axis_name="c", subcore_axis_name="s", num_cores=NC, num_subcores=NS
)


@functools.partial(
    pl.kernel,
    out_type=jax.ShapeDtypeStruct((NPAD * D,), jnp.float32),
    mesh=_mesh,
    compiler_params=pltpu.CompilerParams(needs_layout_passes=False),
    scratch_types=[
        pltpu.VMEM(((NPW + 1) * D,), jnp.float32),
        pltpu.VMEM(((NPW + 1) * D,), jnp.float32),
        pltpu.VMEM((NPW + 2 * L,), jnp.float32),
        pltpu.VMEM((C,), jnp.int32),
        pltpu.VMEM((C,), jnp.int32),
        pltpu.VMEM((MBUF,), jnp.int32),
        pltpu.VMEM((NSLOT * G,), jnp.int32),
        pltpu.VMEM((NSLOT * G, D), jnp.float32),
        pltpu.VMEM((L,), jnp.float32),
        pltpu.SemaphoreType.DMA,
        pltpu.SemaphoreType.DMA,
        pltpu.SemaphoreType.DMA,
        pltpu.SemaphoreType.DMA,
    ],
)
def _mp(z_hbm, ep_hbm, w_hbm, out_hbm,
        s_acc, m_acc, cnt_acc, chA, chB, mpack, gidx, rows, zv,
        semA, semB, sem0, sem1):
    sems = [sem0, sem1]
    cid = lax.axis_index("c")
    sid = lax.axis_index("s")
    wid = cid * NS + sid
    lo = wid * NPW

    fzeros = jnp.zeros((L,), jnp.float32)
    fones = jnp.ones((L,), jnp.float32)
    izeros = jnp.zeros((L,), jnp.int32)
    iones = jnp.ones((L,), jnp.int32)
    dumpv = jnp.full((L,), DUMP << 14, jnp.int32)
    lowv = jnp.full((L,), 16383, jnp.int32)
    highm = jnp.full((L,), -65536, jnp.int32)
    sh16 = jnp.full((L,), 16, jnp.int32)
    sh14 = jnp.full((L,), 14, jnp.int32)
    lanes = lax.iota(jnp.int32, L)
    e1vec = jnp.where(lanes == izeros, fones, fzeros)
    lov14 = jnp.full((L,), lo * 16384, jnp.int32)
    hiv14 = jnp.full((L,), (lo + NPW) * 16384, jnp.int32)

    negs = jnp.full((L,), NEG, jnp.float32)

    def init_acc(i, carry):
        s_acc[pl.ds(i * L, L)] = fzeros
        m_acc[pl.ds(i * L, L)] = negs
        return carry
    lax.fori_loop(0, (NPW + 1) * D // L, init_acc, 0)

    def init_cnt(i, carry):
        cnt_acc[pl.ds(i * L, L)] = fzeros
        return carry
    lax.fori_loop(0, (NPW + 2 * L) // L, init_cnt, 0)

    def init_mp(i, carry):
        mpack[pl.ds(i * L, L)] = dumpv
        return carry
    lax.fori_loop(0, MBUF // L, init_mp, 0)

    pltpu.sync_copy(z_hbm, zv)

    def issue_chunk(k, cb, sem):
        pltpu.async_copy(ep_hbm.at[pl.ds(k * C, C)], cb, sem)

    def wait_chunk(cb, sem):
        pltpu.make_async_copy(ep_hbm.at[pl.ds(0, C)], cb, sem).wait()

    def fire(b, s):
        g0 = s * G
        for kk in range(G // L):
            pv = mpack[pl.ds(b * G + kk * L, L)]
            gidx[pl.ds(g0 + kk * L, L)] = pv & lowv
        pltpu.async_copy(
            w_hbm.at[gidx.at[pl.ds(g0, G)]],
            rows.at[pl.ds(g0, G), :], sems[s])

    def drain(s):
        g0 = s * G
        pltpu.make_async_copy(
            w_hbm.at[gidx.at[pl.ds(g0, G)]],
            rows.at[pl.ds(g0, G), :], sems[s]).wait()

    def process(b, s):
        e0 = b * G
        r0 = s * G

        def edge_body(t, carry):
            lsp = mpack[pl.ds(e0 + t, L)][0]
            ls = lax.shift_right_logical(lsp, 14)
            base = ls * D
            plsc.addupdate(cnt_acc.at[pl.ds(ls, L)], e1vec)
            for j in range(D // L):
                r = rows[r0 + t, pl.ds(j * L, L)]
                plsc.addupdate(s_acc.at[pl.ds(base + j * L, L)], r)
                mv = m_acc[pl.ds(base + j * L, L)]
                m_acc[pl.ds(base + j * L, L)] = jnp.maximum(mv, r)
            return carry
        lax.fori_loop(0, G, edge_body, 0)

    def process_chunk(cb):
        def scan_body(i, off):
            ev = cb[pl.ds(i * L, L)]
            msk = (ev >= lov14) & (ev < hiv14)
            inc = jnp.where(msk, iones, izeros)
            pos = plsc.cumsum(inc)
            idx = jnp.full((L,), off - 1, jnp.int32) + pos
            plsc.store_scatter(mpack, [idx], ev - lov14, mask=msk)
            pc = plsc.all_reduce_population_count(msk)
            return off + pc[0]
        m = lax.fori_loop(0, C // L, scan_body, 0)

        nb = (m + (G - 1)) >> 6
        m64 = nb << 6
        mal = (m >> 4) << 4
        mvec = jnp.full((L,), m, jnp.int32)
        for k in range(G // L + 1):
            ab = mal + k * L

            @pl.when(ab < m64)
            def _():
                v = mpack[pl.ds(ab, L)]
                posv = jnp.full((L,), ab, jnp.int32) + lanes
                mpack[pl.ds(ab, L)] = jnp.where(posv >= mvec, dumpv, v)

        for s in range(NSLOT):
            @pl.when(s < nb)
            def _(s=s):
                fire(s, s)

        nq = (nb + (NSLOT - 1)) >> 2

        def quad_body(q, carry):
            for s in range(NSLOT):
                b = q * NSLOT + s

                @pl.when(b < nb)
                def _(b=b, s=s):
                    drain(s)
                    process(b, s)

                    @pl.when(b + NSLOT < nb)
                    def _():
                        fire(b + NSLOT, s)
            return carry
        lax.fori_loop(0, nq, quad_body, 0)

    issue_chunk(0, chA, semA)

    def chunk_pair(p, carry):
        k0 = 2 * p
        wait_chunk(chA, semA)
        issue_chunk(k0 + 1, chB, semB)
        process_chunk(chA)
        wait_chunk(chB, semB)

        @pl.when(k0 + 2 < NCHUNK)
        def _():
            issue_chunk(k0 + 2, chA, semA)
        process_chunk(chB)
        return carry
    lax.fori_loop(0, NCHUNK // 2, chunk_pair, 0)

    zvec = zv[pl.ds(0, L)]
    z0v = jnp.full((L,), zvec[0])
    z1v = jnp.full((L,), zvec[1])
    z2v = jnp.full((L,), zvec[2])

    def comb_group(ng, carry):
        n0 = ng * L
        cv = cnt_acc[pl.ds(n0, L)]
        scalev = z0v + z1v / jnp.maximum(cv, fones)
        zmxv = jnp.where(cv > fzeros, z2v, fzeros)

        for t in range(L):
            sc = jnp.full((L,), scalev[t])
            zm = jnp.full((L,), zmxv[t])
            base = (n0 + t) * D
            for j in range(D // L):
                sj = s_acc[pl.ds(base + j * L, L)]
                mj = m_acc[pl.ds(base + j * L, L)]
                s_acc[pl.ds(base + j * L, L)] = sj * sc + zm * mj
        return carry
    lax.fori_loop(0, NPW // L, comb_group, 0)

    pltpu.sync_copy(s_acc.at[pl.ds(0, NPW * D)],
                    out_hbm.at[pl.ds(wid * (NPW * D), NPW * D)])


def kernel(z_agg_hard, edge_index, x):
    z = jnp.pad(z_agg_hard.reshape(3).astype(jnp.float32), (0, L - 3))
    src = edge_index[0].astype(jnp.int32)
    dst = edge_index[1].astype(jnp.int32)
    epack = lax.shift_left(src, 14) | dst
    out = _mp(z, epack, x)
    return out.reshape(NPAD, D)[:N]
